# Initial kernel scaffold; baseline (speedup 1.0000x reference)
import jax
import jax.numpy as jnp
from jax.experimental import pallas as pl


def kernel(imgs_in, tp_mu, tx_mu, ty_mu, tw_mu, th_mu, ix, iy):
    raise NotImplementedError("write your pallas kernel here")


# speedup vs baseline: 1.9253x; 1.9253x over previous
import functools

import jax
import jax.numpy as jnp
from jax import lax
from jax.experimental import pallas as pl
from jax.experimental.pallas import tpu as pltpu
from jax.experimental.pallas import tpu_sc as plsc

SIZE_MIN = 10.0
SIZE_MAX = 40.0
ALPHA = 1.0
W_OVER_NW = 4.0
H_OVER_NH = 4.0
PROB_CORR = 0.5
OVERLAP_THR = 0.2
N_OBJ_MAX = 6
N_BOX = 20000
BATCH = 4
IMG = 512

NCORES = 2
NSUB = 16
NW = NCORES * NSUB
LANES = 16
ITEMS = N_BOX * BATCH
PER_TILE = 2560
ITEMS_PAD = PER_TILE * NW
CHUNK = 128

NP = 32768
ROWS = NP // 128


def _sc_tot_body(cum_hbm, bx_hbm, by_hbm, bw_hbm, bh_hbm, out_hbm,
                 bxv, byv, bwv, bhv, idxv, gatv, totv, sem):
    wid = lax.axis_index("s") * NCORES + lax.axis_index("c")
    base = wid * PER_TILE

    pltpu.sync_copy(bx_hbm.at[pl.ds(base, PER_TILE)], bxv)
    pltpu.sync_copy(by_hbm.at[pl.ds(base, PER_TILE)], byv)
    pltpu.sync_copy(bw_hbm.at[pl.ds(base, PER_TILE)], bwv)
    pltpu.sync_copy(bh_hbm.at[pl.ds(base, PER_TILE)], bhv)

    def idx_body(i, carry):
        sl = pl.ds(i * LANES, LANES)
        bx = bxv[sl]
        by = byv[sl]
        bw = bwv[sl]
        bh = bhv[sl]
        x1 = jnp.clip((bx - 0.5 * bw).astype(jnp.int32), 0, IMG - 1)
        x3 = jnp.clip((bx + 0.5 * bw).astype(jnp.int32), 0, IMG - 1)
        y1 = jnp.clip((by - 0.5 * bh).astype(jnp.int32), 0, IMG - 1)
        y3 = jnp.clip((by + 0.5 * bh).astype(jnp.int32), 0, IMG - 1)
        gid = base + i * LANES + lax.iota(jnp.int32, LANES)
        boff = (gid & (BATCH - 1)) * (IMG * IMG)
        idxv[pl.ds(0 * PER_TILE + i * LANES, LANES)] = boff + x3 * IMG + y3
        idxv[pl.ds(1 * PER_TILE + i * LANES, LANES)] = boff + x1 * IMG + y1
        idxv[pl.ds(2 * PER_TILE + i * LANES, LANES)] = boff + x1 * IMG + y3
        idxv[pl.ds(3 * PER_TILE + i * LANES, LANES)] = boff + x3 * IMG + y1
        return carry

    lax.fori_loop(0, PER_TILE // LANES, idx_body, 0)

    copies = []
    for c in range(4 * PER_TILE // CHUNK):
        sl = pl.ds(c * CHUNK, CHUNK)
        copies.append(pltpu.async_copy(cum_hbm.at[idxv.at[sl]], gatv.at[sl], sem))
    for cp in copies:
        cp.wait()

    def tot_body(i, carry):
        sl = pl.ds(i * LANES, LANES)
        c33 = gatv[pl.ds(0 * PER_TILE + i * LANES, LANES)]
        c11 = gatv[pl.ds(1 * PER_TILE + i * LANES, LANES)]
        c13 = gatv[pl.ds(2 * PER_TILE + i * LANES, LANES)]
        c31 = gatv[pl.ds(3 * PER_TILE + i * LANES, LANES)]
        totv[sl] = ((c33 + c11) - c13) - c31
        return carry

    lax.fori_loop(0, PER_TILE // LANES, tot_body, 0)

    pltpu.sync_copy(totv, out_hbm.at[pl.ds(base, PER_TILE)])


@functools.cache
def _sc_tot():
    return pl.kernel(
        _sc_tot_body,
        out_type=jax.ShapeDtypeStruct((ITEMS_PAD,), jnp.float32),
        mesh=plsc.VectorSubcoreMesh(core_axis_name="c", subcore_axis_name="s"),
        scratch_types=[
            pltpu.VMEM((PER_TILE,), jnp.float32),
            pltpu.VMEM((PER_TILE,), jnp.float32),
            pltpu.VMEM((PER_TILE,), jnp.float32),
            pltpu.VMEM((PER_TILE,), jnp.float32),
            pltpu.VMEM((4 * PER_TILE,), jnp.int32),
            pltpu.VMEM((4 * PER_TILE,), jnp.float32),
            pltpu.VMEM((PER_TILE,), jnp.float32),
            pltpu.SemaphoreType.DMA,
        ],
    )


def _make_tc_body(rows, n_real, n_obj):
    total = rows * 128
    n_log2 = total.bit_length() - 1
    assert (1 << n_log2) == total

    def tc_body(av_ref, sig_ref, bx_ref, by_ref, bw_ref, bh_ref, out_ref):
        av = av_ref[0]
        sig = sig_ref[0]
        bxs = bx_ref[0]
        bys = by_ref[0]
        bws = bw_ref[0]
        bhs = bh_ref[0]

        ri = lax.broadcasted_iota(jnp.int32, (rows, 128), 0)
        li = lax.broadcasted_iota(jnp.int32, (rows, 128), 1)
        fid = ri * 128 + li

        def bitonic_sort(arrs, lt_fn):
            def stage(a, d, k, axis):
                if axis == 0:
                    dd = d >> 7
                    n_ax = rows
                else:
                    dd = d
                    n_ax = 128
                low = (fid & d) == 0
                asc = (fid & k) == 0
                ps = []
                for x in a:
                    dn = pltpu.roll(x, n_ax - dd, axis)
                    up = pltpu.roll(x, dd, axis)
                    ps.append(jnp.where(low, dn, up))
                ltm = lt_fn(ps, a)
                take_p = (asc == low) == ltm
                return tuple(jnp.where(take_p, p, x) for p, x in zip(ps, a))

            def round_fn(kk, a):
                k = 1 << kk
                n_row = jnp.maximum(kk - 7, 0)

                def row_body(t, a):
                    return stage(a, k >> (t + 1), k, 0)

                a = lax.fori_loop(0, n_row, row_body, a)
                n_lane = jnp.minimum(kk, 7)

                def lane_body(t, a):
                    return stage(a, 1 << (n_lane - 1 - t), k, 1)

                return lax.fori_loop(0, n_lane, lane_body, a)

            return lax.fori_loop(1, n_log2 + 1, round_fn, arrs)

        def lt_key(ps, xs):
            return ps[0] < xs[0]

        _, sorted_idx = bitonic_sort((av, fid), lt_key)
        packed = bitonic_sort((sorted_idx * total + fid,), lt_key)[0]
        rank = packed & (total - 1)

        t = (rank.astype(jnp.float32) + 1.0) / jnp.float32(n_real + 1)
        t2 = t * t
        t4 = t2 * t2
        t8 = t4 * t4
        papp = t8 * t2
        p = (1.0 - PROB_CORR) * sig + PROB_CORR * papp

        active = (fid < n_real).astype(jnp.float32)

        ri8 = lax.broadcasted_iota(jnp.int32, (8, 128), 0)
        li8 = lax.broadcasted_iota(jnp.int32, (8, 128), 1)
        res = jnp.zeros((8, 128), jnp.float32)

        for k in range(n_obj):
            score = p * active - 1e9 * (1.0 - active)
            m = jnp.max(score)
            idx = jnp.min(jnp.where(score == m, fid, jnp.int32(2 ** 30)))
            wmask = fid == idx
            wp = jnp.sum(jnp.where(wmask, p, 0.0))
            wbx = jnp.sum(jnp.where(wmask, bxs, 0.0))
            wby = jnp.sum(jnp.where(wmask, bys, 0.0))
            wbw = jnp.sum(jnp.where(wmask, bws, 0.0))
            wbh = jnp.sum(jnp.where(wmask, bhs, 0.0))

            xx1 = jnp.maximum(bxs - 0.5 * bws, wbx - 0.5 * wbw)
            yy1 = jnp.maximum(bys - 0.5 * bhs, wby - 0.5 * wbh)
            xx2 = jnp.minimum(bxs + 0.5 * bws, wbx + 0.5 * wbw)
            yy2 = jnp.minimum(bys + 0.5 * bhs, wby + 0.5 * wbh)
            inter = jnp.maximum(xx2 - xx1, 0.0) * jnp.maximum(yy2 - yy1, 0.0)
            union = bws * bhs + wbw * wbh - inter
            ov = inter / jnp.maximum(union, 1e-8)
            active = active * (ov <= OVERLAP_THR).astype(jnp.float32)

            for c, val in enumerate((wp, wbx, wby, wbw, wbh)):
                res = jnp.where((ri8 == k) & (li8 == c), val, res)

        out_ref[0] = res

    return tc_body


def _tc_call(av_tc, sig_tc, bx_tc, by_tc, bw_tc, bh_tc, rows, n_real, n_obj,
             interpret=False):
    nb = av_tc.shape[0]
    spec = pl.BlockSpec((1, rows, 128), lambda b: (b, 0, 0))
    return pl.pallas_call(
        _make_tc_body(rows, n_real, n_obj),
        grid=(nb,),
        in_specs=[spec] * 6,
        out_specs=pl.BlockSpec((1, 8, 128), lambda b: (b, 0, 0)),
        out_shape=jax.ShapeDtypeStruct((nb, 8, 128), jnp.float32),
        compiler_params=pltpu.CompilerParams(
            dimension_semantics=("parallel",)),
        interpret=interpret,
    )(av_tc, sig_tc, bx_tc, by_tc, bw_tc, bh_tc)


def _prep_tc(x, pad_value, rows):
    nb = x.shape[1]
    xt = jnp.transpose(x, (1, 0))
    xt = jnp.pad(xt, ((0, 0), (0, rows * 128 - x.shape[0])),
                 constant_values=pad_value)
    return xt.reshape(nb, rows, 128)


def kernel(imgs_in, tp_mu, tx_mu, ty_mu, tw_mu, th_mu, ix, iy):
    ixf = ix.astype(jnp.float32)
    iyf = iy.astype(jnp.float32)
    prob_before = jax.nn.sigmoid(tp_mu)
    bx = W_OVER_NW * (ixf + jax.nn.sigmoid(ALPHA * tx_mu))
    by = H_OVER_NH * (iyf + jax.nn.sigmoid(ALPHA * ty_mu))
    bw = SIZE_MIN + (SIZE_MAX - SIZE_MIN) * jax.nn.sigmoid(ALPHA * tw_mu)
    bh = SIZE_MIN + (SIZE_MAX - SIZE_MIN) * jax.nn.sigmoid(ALPHA * th_mu)

    cum = jnp.sum(jnp.cumsum(jnp.cumsum(imgs_in, axis=-1), axis=-2), axis=-3)

    def flat_pad(x):
        f = x.squeeze(-1).reshape(-1)
        return jnp.pad(f, (0, ITEMS_PAD - ITEMS), constant_values=16.0)

    tot_pad = _sc_tot()(cum.reshape(-1), flat_pad(bx), flat_pad(by),
                        flat_pad(bw), flat_pad(bh))
    tot = tot_pad[:ITEMS].reshape(N_BOX, BATCH)

    area = (bw * bh).squeeze(-1)
    av = tot / area

    av_tc = _prep_tc(av, jnp.inf, ROWS)
    sig_tc = _prep_tc(prob_before.squeeze(-1), 0.0, ROWS)
    bx_tc = _prep_tc(bx.squeeze(-1), 0.0, ROWS)
    by_tc = _prep_tc(by.squeeze(-1), 0.0, ROWS)
    bw_tc = _prep_tc(bw.squeeze(-1), 1.0, ROWS)
    bh_tc = _prep_tc(bh.squeeze(-1), 1.0, ROWS)

    res = _tc_call(av_tc, sig_tc, bx_tc, by_tc, bw_tc, bh_tc,
                   ROWS, N_BOX, N_OBJ_MAX)
    return jnp.transpose(res[:, :N_OBJ_MAX, :5], (1, 0, 2))
